# baseline (speedup 1.0000x reference)
import functools

import jax
import jax.numpy as jnp
from jax.experimental import pallas as pl
from jax.experimental.pallas import tpu as pltpu

_C = 384
_CR = 230
_CIR = _C - _CR


def _mask_cols(m_row):
    c = m_row.shape[1]
    ri = jax.lax.broadcasted_iota(jnp.int32, (c, c), 0)
    ci = jax.lax.broadcasted_iota(jnp.int32, (c, c), 1)
    eyef = (ri == ci).astype(jnp.float32)
    m_col = jax.lax.dot_general(
        eyef, m_row, (((1,), (1,)), ((), ())),
        preferred_element_type=jnp.float32)
    m_j = jnp.broadcast_to(m_row, (c, c))
    m_c = jnp.broadcast_to(m_col, (c, c))
    cmp = (m_j > m_c) | ((m_j == m_c) & (ci < ri))
    rank = jnp.sum(cmp.astype(jnp.float32), axis=1, keepdims=True)
    return rank < float(_CR)


def _reduce_body(x_ref, m_ref, out_ref):
    xb = x_ref[0]
    mask = _mask_cols(m_ref[0])
    neg = jnp.float32(-jnp.inf)
    sum_all = jnp.sum(xb, axis=0, keepdims=True)
    sum_r = jnp.sum(jnp.where(mask, xb, 0.0), axis=0, keepdims=True)
    max_r = jnp.maximum(jnp.max(jnp.where(mask, xb, neg), axis=0, keepdims=True), 0.0)
    max_ir = jnp.maximum(jnp.max(jnp.where(mask, neg, xb), axis=0, keepdims=True), 0.0)
    r_avg = sum_r * jnp.float32(1.0 / _CR)
    ir_avg = (sum_all - sum_r) * jnp.float32(1.0 / _CIR)
    out_ref[0] = jnp.concatenate([r_avg, max_r, ir_avg, max_ir], axis=0)


def _conv_bn_body(red_ref, w_ref, g_ref, b_ref, out_ref):
    red = red_ref[...]
    b_n, _, h, w = red.shape
    padded = jnp.pad(red, ((0, 0), (0, 0), (3, 3), (3, 3)))

    def conv2(c0):
        acc = jnp.zeros((b_n, h, w), jnp.float32)
        for ci in range(2):
            for ky in range(7):
                for kx in range(7):
                    acc = acc + w_ref[ci * 49 + ky * 7 + kx] * padded[
                        :, c0 + ci, ky:ky + h, kx:kx + w]
        return acc

    def norm_act(y):
        n = y.size
        mean = jnp.sum(y) * (1.0 / n)
        d = y - mean
        var = jnp.sum(d * d) * (1.0 / n)
        yhat = d * jax.lax.rsqrt(var + 1e-5)
        yhat = yhat * g_ref[0] + b_ref[0]
        return jax.nn.sigmoid(jnp.maximum(yhat, 0.0))

    a1 = norm_act(conv2(0))
    a2 = norm_act(conv2(2))
    out_ref[...] = jnp.concatenate([a1[:, None], a2[:, None]], axis=1)


def _apply_body(x_ref, a_ref, m_ref, out_ref):
    xb = x_ref[0]
    mask = _mask_cols(m_ref[0])
    a1 = a_ref[0, 0:1, :]
    a2 = a_ref[0, 1:2, :]
    out_ref[0] = xb * jnp.where(mask, a1, a2)


@jax.jit
def kernel(x, M, conv_w, bn_gamma, bn_beta):
    b_n, c, h, w = x.shape
    hw = h * w
    nblk = 8
    lblk = hw // nblk

    x3 = x.reshape(b_n, c, hw)
    m3 = M.reshape(b_n, 1, c)

    red = pl.pallas_call(
        _reduce_body,
        grid=(b_n, nblk),
        in_specs=[
            pl.BlockSpec((1, c, lblk), lambda b, l: (b, 0, l)),
            pl.BlockSpec((1, 1, c), lambda b, l: (b, 0, 0)),
        ],
        out_specs=pl.BlockSpec((1, 4, lblk), lambda b, l: (b, 0, l)),
        out_shape=jax.ShapeDtypeStruct((b_n, 4, hw), jnp.float32),
    )(x3, m3)

    a_maps = pl.pallas_call(
        _conv_bn_body,
        in_specs=[
            pl.BlockSpec(memory_space=pltpu.VMEM),
            pl.BlockSpec(memory_space=pltpu.SMEM),
            pl.BlockSpec(memory_space=pltpu.SMEM),
            pl.BlockSpec(memory_space=pltpu.SMEM),
        ],
        out_specs=pl.BlockSpec(memory_space=pltpu.VMEM),
        out_shape=jax.ShapeDtypeStruct((b_n, 2, h, w), jnp.float32),
    )(red.reshape(b_n, 4, h, w), conv_w.reshape(-1), bn_gamma, bn_beta)

    out = pl.pallas_call(
        _apply_body,
        grid=(b_n, nblk),
        in_specs=[
            pl.BlockSpec((1, c, lblk), lambda b, l: (b, 0, l)),
            pl.BlockSpec((1, 2, lblk), lambda b, l: (b, 0, l)),
            pl.BlockSpec((1, 1, c), lambda b, l: (b, 0, 0)),
        ],
        out_specs=pl.BlockSpec((1, c, lblk), lambda b, l: (b, 0, l)),
        out_shape=jax.ShapeDtypeStruct((b_n, c, hw), jnp.float32),
    )(x3, a_maps.reshape(b_n, 2, hw), m3)

    return out.reshape(b_n, c, h, w)


# speedup vs baseline: 1.0464x; 1.0464x over previous
import functools

import jax
import jax.numpy as jnp
from jax.experimental import pallas as pl
from jax.experimental.pallas import tpu as pltpu

_C = 384
_CR = 230
_CIR = _C - _CR


def _masks(m_row):
    c = m_row.shape[1]
    ri = jax.lax.broadcasted_iota(jnp.int32, (c, c), 0)
    ci = jax.lax.broadcasted_iota(jnp.int32, (c, c), 1)
    eyef = (ri == ci).astype(jnp.float32)
    m_col = jax.lax.dot_general(
        eyef, m_row, (((1,), (1,)), ((), ())),
        preferred_element_type=jnp.float32)
    m_j = jnp.broadcast_to(m_row, (c, c))
    m_c = jnp.broadcast_to(m_col, (c, c))
    cmp = (m_c > m_j) | ((m_c == m_j) & (ri < ci))
    rank_row = jnp.sum(cmp.astype(jnp.float32), axis=0, keepdims=True)
    maskf_row = (rank_row < float(_CR)).astype(jnp.float32)
    mask_col = jax.lax.dot_general(
        eyef, maskf_row, (((1,), (1,)), ((), ())),
        preferred_element_type=jnp.float32) > 0.5
    return maskf_row, mask_col


def _reduce_body(x_ref, m_ref, out_ref):
    xb = x_ref[0]
    maskf_row, mask_col = _masks(m_ref[0])
    xm = jnp.where(mask_col, xb, 0.0)
    xir = xb - xm
    max_r = jnp.max(xm, axis=0, keepdims=True)
    max_ir = jnp.max(xir, axis=0, keepdims=True)
    w2 = jnp.concatenate(
        [maskf_row, jnp.ones_like(maskf_row)], axis=0)
    sums = jax.lax.dot_general(
        w2, xb, (((1,), (0,)), ((), ())),
        preferred_element_type=jnp.float32)
    sum_r = sums[0:1, :]
    r_avg = sum_r * jnp.float32(1.0 / _CR)
    ir_avg = (sums[1:2, :] - sum_r) * jnp.float32(1.0 / _CIR)
    out_ref[0] = jnp.concatenate([r_avg, max_r, ir_avg, max_ir], axis=0)


def _conv_bn_body(red_ref, w_ref, g_ref, b_ref, out_ref):
    red = red_ref[...]
    b_n, _, h, w = red.shape
    padded = jnp.pad(red, ((0, 0), (0, 0), (3, 3), (3, 3)))

    def conv2(c0):
        acc = jnp.zeros((b_n, h, w), jnp.float32)
        for ci in range(2):
            for ky in range(7):
                for kx in range(7):
                    acc = acc + w_ref[ci * 49 + ky * 7 + kx] * padded[
                        :, c0 + ci, ky:ky + h, kx:kx + w]
        return acc

    def norm_act(y):
        n = y.size
        mean = jnp.sum(y) * (1.0 / n)
        d = y - mean
        var = jnp.sum(d * d) * (1.0 / n)
        yhat = d * jax.lax.rsqrt(var + 1e-5)
        yhat = yhat * g_ref[0] + b_ref[0]
        return jax.nn.sigmoid(jnp.maximum(yhat, 0.0))

    a1 = norm_act(conv2(0))
    a2 = norm_act(conv2(2))
    out_ref[...] = jnp.concatenate([a1[:, None], a2[:, None]], axis=1)


def _apply_body(x_ref, a_ref, m_ref, out_ref):
    xb = x_ref[0]
    _, mask_col = _masks(m_ref[0])
    a1 = a_ref[0, 0:1, :]
    a2 = a_ref[0, 1:2, :]
    out_ref[0] = xb * jnp.where(mask_col, a1, a2)


@jax.jit
def kernel(x, M, conv_w, bn_gamma, bn_beta):
    b_n, c, h, w = x.shape
    hw = h * w
    nblk = 8
    lblk = hw // nblk

    x3 = x.reshape(b_n, c, hw)
    m3 = M.reshape(b_n, 1, c)

    red = pl.pallas_call(
        _reduce_body,
        grid=(b_n, nblk),
        in_specs=[
            pl.BlockSpec((1, c, lblk), lambda b, l: (b, 0, l)),
            pl.BlockSpec((1, 1, c), lambda b, l: (b, 0, 0)),
        ],
        out_specs=pl.BlockSpec((1, 4, lblk), lambda b, l: (b, 0, l)),
        out_shape=jax.ShapeDtypeStruct((b_n, 4, hw), jnp.float32),
    )(x3, m3)

    a_maps = pl.pallas_call(
        _conv_bn_body,
        in_specs=[
            pl.BlockSpec(memory_space=pltpu.VMEM),
            pl.BlockSpec(memory_space=pltpu.SMEM),
            pl.BlockSpec(memory_space=pltpu.SMEM),
            pl.BlockSpec(memory_space=pltpu.SMEM),
        ],
        out_specs=pl.BlockSpec(memory_space=pltpu.VMEM),
        out_shape=jax.ShapeDtypeStruct((b_n, 2, h, w), jnp.float32),
    )(red.reshape(b_n, 4, h, w), conv_w.reshape(-1), bn_gamma, bn_beta)

    out = pl.pallas_call(
        _apply_body,
        grid=(b_n, nblk),
        in_specs=[
            pl.BlockSpec((1, c, lblk), lambda b, l: (b, 0, l)),
            pl.BlockSpec((1, 2, lblk), lambda b, l: (b, 0, l)),
            pl.BlockSpec((1, 1, c), lambda b, l: (b, 0, 0)),
        ],
        out_specs=pl.BlockSpec((1, c, lblk), lambda b, l: (b, 0, l)),
        out_shape=jax.ShapeDtypeStruct((b_n, c, hw), jnp.float32),
    )(x3, a_maps.reshape(b_n, 2, hw), m3)

    return out.reshape(b_n, c, h, w)


# speedup vs baseline: 1.0470x; 1.0005x over previous
import jax
import jax.numpy as jnp
from jax.experimental import pallas as pl
from jax.experimental.pallas import tpu as pltpu

_C = 384
_CR = 230
_CIR = _C - _CR


def _masks(m_row):
    c = m_row.shape[1]
    ri = jax.lax.broadcasted_iota(jnp.int32, (c, c), 0)
    ci = jax.lax.broadcasted_iota(jnp.int32, (c, c), 1)
    eyef = (ri == ci).astype(jnp.float32)
    m_col = jax.lax.dot_general(
        eyef, m_row, (((1,), (1,)), ((), ())),
        preferred_element_type=jnp.float32)
    m_j = jnp.broadcast_to(m_row, (c, c))
    m_c = jnp.broadcast_to(m_col, (c, c))
    cmp = (m_c > m_j) | ((m_c == m_j) & (ri < ci))
    rank_row = jnp.sum(cmp.astype(jnp.float32), axis=0, keepdims=True)
    maskf_row = (rank_row < float(_CR)).astype(jnp.float32)
    mask_col = jax.lax.dot_general(
        eyef, maskf_row, (((1,), (1,)), ((), ())),
        preferred_element_type=jnp.float32) > 0.5
    return maskf_row, mask_col


def _reduce_body(x_ref, m_ref, out_ref):
    xb = x_ref[0]
    maskf_row, mask_col = _masks(m_ref[0])
    xm = jnp.where(mask_col, xb, 0.0)
    xir = xb - xm
    max_r = jnp.max(xm, axis=0, keepdims=True)
    max_ir = jnp.max(xir, axis=0, keepdims=True)
    w2 = jnp.concatenate(
        [maskf_row, jnp.ones_like(maskf_row)], axis=0)
    sums = jax.lax.dot_general(
        w2, xb, (((1,), (0,)), ((), ())),
        preferred_element_type=jnp.float32)
    sum_r = sums[0:1, :]
    r_avg = sum_r * jnp.float32(1.0 / _CR)
    ir_avg = (sums[1:2, :] - sum_r) * jnp.float32(1.0 / _CIR)
    out_ref[0] = jnp.concatenate([r_avg, max_r, ir_avg, max_ir], axis=0)


def _conv_bn_body(red_ref, w_ref, g_ref, b_ref, out_ref):
    red = red_ref[...]
    b_n, _, h, w = red.shape
    padded = jnp.pad(red, ((0, 0), (0, 0), (3, 3), (3, 3)))

    def conv2(c0):
        acc = jnp.zeros((b_n, h, w), jnp.float32)
        for ci in range(2):
            for ky in range(7):
                for kx in range(7):
                    acc = acc + w_ref[ci * 49 + ky * 7 + kx] * padded[
                        :, c0 + ci, ky:ky + h, kx:kx + w]
        return acc

    def norm_act(y):
        n = y.size
        mean = jnp.sum(y) * (1.0 / n)
        d = y - mean
        var = jnp.sum(d * d) * (1.0 / n)
        yhat = d * jax.lax.rsqrt(var + 1e-5)
        yhat = yhat * g_ref[0] + b_ref[0]
        return jax.nn.sigmoid(jnp.maximum(yhat, 0.0))

    a1 = norm_act(conv2(0))
    a2 = norm_act(conv2(2))
    out_ref[...] = jnp.concatenate([a1[:, None], a2[:, None]], axis=1)


_LBLK = 3584
_NBLK = 50176 // _LBLK
_IND = 4
_OUTD = 3


def _apply_body(x_hbm, a_ref, m_ref, out_hbm,
                i0, i1, i2, i3, o0, o1, o2,
                si0, si1, si2, si3, so0, so1, so2):
    ibufs = [i0, i1, i2, i3]
    obufs = [o0, o1, o2]
    isems = [si0, si1, si2, si3]
    osems = [so0, so1, so2]
    b_n = x_hbm.shape[0]
    nstep = b_n * _NBLK

    mask_cols = [_masks(m_ref[b])[1] for b in range(b_n)]

    def src(i):
        b, j = i // _NBLK, i % _NBLK
        return x_hbm.at[b, :, pl.ds(j * _LBLK, _LBLK)]

    def dst(i):
        b, j = i // _NBLK, i % _NBLK
        return out_hbm.at[b, :, pl.ds(j * _LBLK, _LBLK)]

    in_h = {}
    out_h = {}
    for i in range(_IND):
        in_h[i] = pltpu.async_copy(src(i), ibufs[i % _IND], isems[i % _IND])
    for i in range(nstep):
        b, j = i // _NBLK, i % _NBLK
        in_h[i].wait()
        if i >= _OUTD:
            out_h[i - _OUTD].wait()
        a1 = a_ref[b, 0:1, pl.ds(j * _LBLK, _LBLK)]
        a2 = a_ref[b, 1:2, pl.ds(j * _LBLK, _LBLK)]
        obufs[i % _OUTD][...] = ibufs[i % _IND][...] * jnp.where(
            mask_cols[b], a1, a2)
        out_h[i] = pltpu.async_copy(obufs[i % _OUTD], dst(i), osems[i % _OUTD])
        if i + _IND < nstep:
            in_h[i + _IND] = pltpu.async_copy(
                src(i + _IND), ibufs[(i + _IND) % _IND], isems[(i + _IND) % _IND])
    for i in range(nstep - _OUTD, nstep):
        out_h[i].wait()


@jax.jit
def kernel(x, M, conv_w, bn_gamma, bn_beta):
    b_n, c, h, w = x.shape
    hw = h * w
    nblk = 8
    lblk = hw // nblk

    x3 = x.reshape(b_n, c, hw)
    m3 = M.reshape(b_n, 1, c)

    red = pl.pallas_call(
        _reduce_body,
        grid=(b_n, nblk),
        in_specs=[
            pl.BlockSpec((1, c, lblk), lambda b, l: (b, 0, l)),
            pl.BlockSpec((1, 1, c), lambda b, l: (b, 0, 0)),
        ],
        out_specs=pl.BlockSpec((1, 4, lblk), lambda b, l: (b, 0, l)),
        out_shape=jax.ShapeDtypeStruct((b_n, 4, hw), jnp.float32),
    )(x3, m3)

    a_maps = pl.pallas_call(
        _conv_bn_body,
        in_specs=[
            pl.BlockSpec(memory_space=pltpu.MemorySpace.VMEM),
            pl.BlockSpec(memory_space=pltpu.MemorySpace.SMEM),
            pl.BlockSpec(memory_space=pltpu.MemorySpace.SMEM),
            pl.BlockSpec(memory_space=pltpu.MemorySpace.SMEM),
        ],
        out_specs=pl.BlockSpec(memory_space=pltpu.MemorySpace.VMEM),
        out_shape=jax.ShapeDtypeStruct((b_n, 2, h, w), jnp.float32),
    )(red.reshape(b_n, 4, h, w), conv_w.reshape(-1), bn_gamma, bn_beta)

    out = pl.pallas_call(
        _apply_body,
        in_specs=[
            pl.BlockSpec(memory_space=pltpu.MemorySpace.HBM),
            pl.BlockSpec(memory_space=pltpu.MemorySpace.VMEM),
            pl.BlockSpec(memory_space=pltpu.MemorySpace.VMEM),
        ],
        out_specs=pl.BlockSpec(memory_space=pltpu.MemorySpace.HBM),
        out_shape=jax.ShapeDtypeStruct((b_n, c, hw), jnp.float32),
        scratch_shapes=[pltpu.VMEM((c, _LBLK), jnp.float32)] * (_IND + _OUTD)
        + [pltpu.SemaphoreType.DMA] * (_IND + _OUTD),
        compiler_params=pltpu.CompilerParams(
            vmem_limit_bytes=100 * 1024 * 1024),
    )(x3, a_maps.reshape(b_n, 2, hw), m3)

    return out.reshape(b_n, c, h, w)
